# baseline (speedup 1.0000x reference)
import functools

import jax
import jax.numpy as jnp
from jax import lax
from jax.experimental import pallas as pl
from jax.experimental.pallas import tpu as pltpu
from jax.experimental.pallas import tpu_sc as plsc

B = 16384
F = 26
V = 100000
D = 16
H = 200

NC = 2
NS = 16
NW = NC * NS

G = 128
NG = (B * F) // G
GPW = NG // NW
NCHUNK = 4
GPC = GPW // NCHUNK


def _sc_gather(flat_idx, tab2d, w_flat):
    mesh = plsc.VectorSubcoreMesh(core_axis_name="c", subcore_axis_name="s")

    @functools.partial(
        pl.kernel,
        mesh=mesh,
        compiler_params=pltpu.CompilerParams(use_tc_tiling_on_sc=False),
        out_type=[
            jax.ShapeDtypeStruct((NG, G, D), jnp.float32),
            jax.ShapeDtypeStruct((NG, G), jnp.float32),
        ],
        scratch_types=[
            pltpu.VMEM((GPW, G), jnp.int32),
            pltpu.VMEM((G, D), jnp.float32),
            pltpu.VMEM((G,), jnp.float32),
            pltpu.SemaphoreType.DMA,
            pltpu.SemaphoreType.DMA,
        ],
    )
    def gather_kernel(idx_hbm, tab_hbm, w_hbm, emb_out, wv_out,
                      idx_v, rows_v, w_v, sem_r, sem_w):
        wid = lax.axis_index("s") * NC + lax.axis_index("c")
        g0 = wid * GPW
        pltpu.sync_copy(idx_hbm.at[pl.ds(g0, GPW)], idx_v)

        def body(j, carry):
            cp_r = pltpu.async_copy(tab_hbm.at[idx_v.at[j]], rows_v, sem_r)
            cp_w = pltpu.async_copy(w_hbm.at[idx_v.at[j]], w_v, sem_w)
            cp_r.wait()
            cp_w.wait()
            pltpu.sync_copy(rows_v, emb_out.at[g0 + j])
            pltpu.sync_copy(w_v, wv_out.at[g0 + j])
            return carry

        lax.fori_loop(0, GPW, body, 0)

    return gather_kernel(flat_idx, tab2d, w_flat)


BLK = 512


def _tc_head(emb, wv, W1, b1, W2, b2, W3, b3, Wd, bd):
    def body(emb_ref, wv_ref, W1_ref, b1_ref, W2_ref, b2_ref, W3_ref,
             b3_ref, Wd_ref, bd_ref, out_ref):
        e = emb_ref[...]
        ri = lax.broadcasted_iota(jnp.int32, (F * D, D), 0)
        ci = lax.broadcasted_iota(jnp.int32, (F * D, D), 1)
        S = jnp.where((ri % D) == ci, 1.0, 0.0).astype(jnp.float32)
        sum_f = jnp.dot(e, S, preferred_element_type=jnp.float32)
        sos = jnp.sum(sum_f * sum_f, axis=1, keepdims=True)
        ssq = jnp.sum(e * e, axis=1, keepdims=True)
        second = 0.5 * (sos - ssq)
        first = jnp.sum(wv_ref[...], axis=1, keepdims=True)
        h = jnp.maximum(
            jnp.dot(e, W1_ref[...], preferred_element_type=jnp.float32)
            + b1_ref[...], 0.0)
        h = jnp.maximum(
            jnp.dot(h, W2_ref[...], preferred_element_type=jnp.float32)
            + b2_ref[...], 0.0)
        h = jnp.maximum(
            jnp.dot(h, W3_ref[...], preferred_element_type=jnp.float32)
            + b3_ref[...], 0.0)
        deep = jnp.dot(h, Wd_ref[...], preferred_element_type=jnp.float32) \
            + bd_ref[...]
        out_ref[...] = jax.nn.sigmoid(first + second + deep)

    return pl.pallas_call(
        body,
        grid=(B // BLK,),
        in_specs=[
            pl.BlockSpec((BLK, F * D), lambda i: (i, 0)),
            pl.BlockSpec((BLK, F), lambda i: (i, 0)),
            pl.BlockSpec((F * D, H), lambda i: (0, 0)),
            pl.BlockSpec((1, H), lambda i: (0, 0)),
            pl.BlockSpec((H, H), lambda i: (0, 0)),
            pl.BlockSpec((1, H), lambda i: (0, 0)),
            pl.BlockSpec((H, H), lambda i: (0, 0)),
            pl.BlockSpec((1, H), lambda i: (0, 0)),
            pl.BlockSpec((H, 1), lambda i: (0, 0)),
            pl.BlockSpec((1, 1), lambda i: (0, 0)),
        ],
        out_specs=pl.BlockSpec((BLK, 1), lambda i: (i, 0)),
        out_shape=jax.ShapeDtypeStruct((B, 1), jnp.float32),
    )(emb, wv, W1, b1.reshape(1, H), W2, b2.reshape(1, H), W3,
      b3.reshape(1, H), Wd, bd.reshape(1, 1))


def kernel(indices, tables, w_fm, W1, b1, W2, b2, W3, b3, Wd, bd):
    offsets = (jnp.arange(F, dtype=jnp.int32) * V)[None, :]
    flat_idx = (indices + offsets).reshape(NG, G)
    tab2d = tables.reshape(F * V, D)
    w_flat = w_fm.reshape(F * V)
    emb3, wv2 = _sc_gather(flat_idx, tab2d, w_flat)
    emb = emb3.reshape(B, F * D)
    wv = wv2.reshape(B, F)
    return _tc_head(emb, wv, W1, b1, W2, b2, W3, b3, Wd, bd)


# speedup vs baseline: 1.0168x; 1.0168x over previous
import functools

import jax
import jax.numpy as jnp
from jax import lax
from jax.experimental import pallas as pl
from jax.experimental.pallas import tpu as pltpu
from jax.experimental.pallas import tpu_sc as plsc

B = 16384
F = 26
V = 100000
D = 16
H = 200

NC = 2
NS = 16
NW = NC * NS

G = 128
NEG = B // 16
EGPW = NEG // NW
NWG = (B * 32) // G
WGPW = NWG // NW


def _sc_gather(i0, i1, i2, i3, iw, tab2d, w_flat):
    mesh = plsc.VectorSubcoreMesh(core_axis_name="c", subcore_axis_name="s")

    @functools.partial(
        pl.kernel,
        mesh=mesh,
        compiler_params=pltpu.CompilerParams(use_tc_tiling_on_sc=False),
        out_type=[
            jax.ShapeDtypeStruct((NEG, G, D), jnp.float32),
            jax.ShapeDtypeStruct((NEG, G, D), jnp.float32),
            jax.ShapeDtypeStruct((NEG, G, D), jnp.float32),
            jax.ShapeDtypeStruct((NEG, G, D), jnp.float32),
            jax.ShapeDtypeStruct((NWG, G), jnp.float32),
        ],
        scratch_types=[
            pltpu.VMEM((4 * EGPW, G), jnp.int32),
            pltpu.VMEM((WGPW, G), jnp.int32),
            pltpu.VMEM((G, D), jnp.float32),
            pltpu.VMEM((G, D), jnp.float32),
            pltpu.VMEM((G,), jnp.float32),
            pltpu.VMEM((G,), jnp.float32),
            pltpu.VMEM((G,), jnp.float32),
            pltpu.VMEM((G,), jnp.float32),
            pltpu.SemaphoreType.DMA,
            pltpu.SemaphoreType.DMA,
            pltpu.SemaphoreType.DMA,
            pltpu.SemaphoreType.DMA,
        ],
    )
    def gather_kernel(i0_h, i1_h, i2_h, i3_h, iw_h, tab_h, w_h,
                      e0_h, e1_h, e2_h, e3_h, wv_h,
                      idxe_v, idxw_v, bufa, bufb, wba, wbb, wbc, wbd,
                      sema, semb, semc, semd):
        wid = lax.axis_index("s") * NC + lax.axis_index("c")
        eb = wid * EGPW
        wb = wid * WGPW
        for q, iq_h in enumerate((i0_h, i1_h, i2_h, i3_h)):
            pltpu.sync_copy(iq_h.at[pl.ds(eb, EGPW)],
                            idxe_v.at[pl.ds(q * EGPW, EGPW)])
        pltpu.sync_copy(iw_h.at[pl.ds(wb, WGPW)], idxw_v)

        for q, eq_h in enumerate((e0_h, e1_h, e2_h, e3_h)):
            def ebody(j, carry, q=q, eq_h=eq_h):
                k = 2 * j
                cpa = pltpu.async_copy(
                    tab_h.at[idxe_v.at[q * EGPW + k]], bufa, sema)
                cpb = pltpu.async_copy(
                    tab_h.at[idxe_v.at[q * EGPW + k + 1]], bufb, semb)
                cpa.wait()
                pltpu.sync_copy(bufa, eq_h.at[eb + k])
                cpb.wait()
                pltpu.sync_copy(bufb, eq_h.at[eb + k + 1])
                return carry
            lax.fori_loop(0, EGPW // 2, ebody, 0)

        def wbody(j, carry):
            g = 4 * j
            cps = []
            for t, (wbuf, sem) in enumerate(
                    ((wba, sema), (wbb, semb), (wbc, semc), (wbd, semd))):
                cps.append(pltpu.async_copy(
                    w_h.at[idxw_v.at[g + t]], wbuf, sem))
            for t, (wbuf, _) in enumerate(
                    ((wba, sema), (wbb, semb), (wbc, semc), (wbd, semd))):
                cps[t].wait()
                pltpu.sync_copy(wbuf, wv_h.at[wb + g + t])
            return carry
        lax.fori_loop(0, WGPW // 4, wbody, 0)

    return gather_kernel(i0, i1, i2, i3, iw, tab2d, w_flat)


BLK = 512


def _tc_head(e0, e1, e2, e3, wv, W1x, b1, W2, b2, W3, b3, Wd, bd):
    def body(e0_ref, e1_ref, e2_ref, e3_ref, wv_ref, W1x_ref, b1_ref,
             W2_ref, b2_ref, W3_ref, b3_ref, Wd_ref, bd_ref, out_ref):
        E0, E1, E2, E3 = e0_ref[...], e1_ref[...], e2_ref[...], e3_ref[...]
        m3 = (lax.broadcasted_iota(jnp.int32, (1, G), 1) < 2 * D).astype(
            jnp.float32)
        E3m = E3 * m3
        rj = lax.broadcasted_iota(jnp.int32, (G, D), 0)
        cd = lax.broadcasted_iota(jnp.int32, (G, D), 1)
        S = jnp.where((rj % D) == cd, 1.0, 0.0).astype(jnp.float32)
        Esum = E0 + E1 + E2 + E3m
        sum_d = jnp.dot(Esum, S, preferred_element_type=jnp.float32)
        sos = jnp.sum(sum_d * sum_d, axis=1, keepdims=True)
        ssq = jnp.sum(E0 * E0 + E1 * E1 + E2 * E2 + E3m * E3m,
                      axis=1, keepdims=True)
        second = 0.5 * (sos - ssq)
        mw = (lax.broadcasted_iota(jnp.int32, (1, 32), 1) < F).astype(
            jnp.float32)
        first = jnp.sum(wv_ref[...] * mw, axis=1, keepdims=True)
        W1x = W1x_ref[...]
        h = (jnp.dot(E0, W1x[0:128], preferred_element_type=jnp.float32)
             + jnp.dot(E1, W1x[128:256], preferred_element_type=jnp.float32)
             + jnp.dot(E2, W1x[256:384], preferred_element_type=jnp.float32)
             + jnp.dot(E3, W1x[384:512], preferred_element_type=jnp.float32))
        h = jnp.maximum(h + b1_ref[...], 0.0)
        h = jnp.maximum(
            jnp.dot(h, W2_ref[...], preferred_element_type=jnp.float32)
            + b2_ref[...], 0.0)
        h = jnp.maximum(
            jnp.dot(h, W3_ref[...], preferred_element_type=jnp.float32)
            + b3_ref[...], 0.0)
        deep = jnp.dot(h, Wd_ref[...], preferred_element_type=jnp.float32) \
            + bd_ref[...]
        out_ref[...] = jax.nn.sigmoid(first + second + deep)

    return pl.pallas_call(
        body,
        grid=(B // BLK,),
        in_specs=[
            pl.BlockSpec((BLK, G), lambda i: (i, 0)),
            pl.BlockSpec((BLK, G), lambda i: (i, 0)),
            pl.BlockSpec((BLK, G), lambda i: (i, 0)),
            pl.BlockSpec((BLK, G), lambda i: (i, 0)),
            pl.BlockSpec((BLK, 32), lambda i: (i, 0)),
            pl.BlockSpec((512, H), lambda i: (0, 0)),
            pl.BlockSpec((1, H), lambda i: (0, 0)),
            pl.BlockSpec((H, H), lambda i: (0, 0)),
            pl.BlockSpec((1, H), lambda i: (0, 0)),
            pl.BlockSpec((H, H), lambda i: (0, 0)),
            pl.BlockSpec((1, H), lambda i: (0, 0)),
            pl.BlockSpec((H, 1), lambda i: (0, 0)),
            pl.BlockSpec((1, 1), lambda i: (0, 0)),
        ],
        out_specs=pl.BlockSpec((BLK, 1), lambda i: (i, 0)),
        out_shape=jax.ShapeDtypeStruct((B, 1), jnp.float32),
    )(e0, e1, e2, e3, wv, W1x, b1.reshape(1, H), W2, b2.reshape(1, H),
      W3, b3.reshape(1, H), Wd, bd.reshape(1, 1))


def kernel(indices, tables, w_fm, W1, b1, W2, b2, W3, b3, Wd, bd):
    offsets = (jnp.arange(F, dtype=jnp.int32) * V)[None, :]
    flat = indices + offsets
    flat32 = jnp.concatenate([flat, flat[:, :32 - F]], axis=1)
    iqs = [flat32[:, 8 * q:8 * q + 8].reshape(NEG, G) for q in range(4)]
    iw = flat32.reshape(NWG, G)
    tab2d = tables.reshape(F * V, D)
    w_flat = w_fm.reshape(F * V)
    E0, E1, E2, E3, WV = _sc_gather(iqs[0], iqs[1], iqs[2], iqs[3], iw,
                                    tab2d, w_flat)
    e0 = E0.reshape(B, G)
    e1 = E1.reshape(B, G)
    e2 = E2.reshape(B, G)
    e3 = E3.reshape(B, G)
    wv = WV.reshape(B, 32)
    W1x = jnp.pad(W1, ((0, 512 - F * D), (0, 0)))
    return _tc_head(e0, e1, e2, e3, wv, W1x, b1, W2, b2, W3, b3, Wd, bd)
